# baseline (speedup 1.0000x reference)
import functools

import jax
import jax.numpy as jnp
from jax import lax
from jax.experimental import pallas as pl
from jax.experimental.pallas import tpu as pltpu
from jax.experimental.pallas import tpu_sc as plsc

EMB = 64
GRP = 128
NUM_WORKERS = 32
MM_BLK = 2048


def _sc_gather_body(s_idx, p_idx, o_idx, node_tab, rel_tab,
                    s_out, p_out, o_out, idx_v, rows_v, sem):
    wid = lax.axis_index("s") * 2 + lax.axis_index("c")
    ngrp_total = s_idx.shape[0]
    ngrp_per_tile = ngrp_total // NUM_WORKERS
    base = wid * ngrp_per_tile

    def one_table(idx_hbm, tab_hbm, out_hbm):
        def body(k, carry):
            g = base + k
            pltpu.sync_copy(idx_hbm.at[g], idx_v)
            pltpu.async_copy(tab_hbm.at[idx_v], rows_v, sem).wait()
            pltpu.sync_copy(rows_v, out_hbm.at[pl.ds(g * GRP, GRP)])
            return carry
        lax.fori_loop(0, ngrp_per_tile, body, 0)

    one_table(s_idx, node_tab, s_out)
    one_table(p_idx, rel_tab, p_out)
    one_table(o_idx, node_tab, o_out)


def _sc_gather(s_idx, p_idx, o_idx, node_tab, rel_tab):
    n = s_idx.shape[0] * GRP
    out_t = jax.ShapeDtypeStruct((n, EMB), jnp.float32)
    mesh = plsc.VectorSubcoreMesh(core_axis_name="c", subcore_axis_name="s")
    f = functools.partial(
        pl.kernel,
        mesh=mesh,
        compiler_params=pltpu.CompilerParams(use_tc_tiling_on_sc=False),
        out_type=(out_t, out_t, out_t),
        scratch_types=[
            pltpu.VMEM((GRP,), jnp.int32),
            pltpu.VMEM((GRP, EMB), jnp.float32),
            pltpu.SemaphoreType.DMA,
        ],
    )(_sc_gather_body)
    return f(s_idx, p_idx, o_idx, node_tab, rel_tab)


def _mm_body(s_ref, p_ref, o_ref, w_ref, b_ref, out_ref):
    w = w_ref[...]
    acc = jnp.dot(s_ref[...], w[0:EMB], preferred_element_type=jnp.float32)
    acc = acc + jnp.dot(p_ref[...], w[EMB:2 * EMB],
                        preferred_element_type=jnp.float32)
    acc = acc + jnp.dot(o_ref[...], w[2 * EMB:3 * EMB],
                        preferred_element_type=jnp.float32)
    out_ref[...] = acc + b_ref[...]


def _mm(s_rows, p_rows, o_rows, w, b2d):
    n = s_rows.shape[0]
    grid = (n // MM_BLK,)
    row_spec = pl.BlockSpec((MM_BLK, EMB), lambda i: (i, 0))
    return pl.pallas_call(
        _mm_body,
        grid=grid,
        in_specs=[
            row_spec, row_spec, row_spec,
            pl.BlockSpec((3 * EMB, EMB), lambda i: (0, 0)),
            pl.BlockSpec((1, EMB), lambda i: (0, 0)),
        ],
        out_specs=row_spec,
        out_shape=jax.ShapeDtypeStruct((n, EMB), jnp.float32),
    )(s_rows, p_rows, o_rows, w, b2d)


def kernel(triples, node_table, relation_table, W, b):
    bsz, esz, _ = triples.shape
    n = bsz * esz
    t = triples.reshape(n, 3).astype(jnp.int32)
    s_idx = t[:, 0].reshape(n // GRP, GRP)
    p_idx = t[:, 1].reshape(n // GRP, GRP)
    o_idx = t[:, 2].reshape(n // GRP, GRP)
    s_rows, p_rows, o_rows = _sc_gather(
        s_idx, p_idx, o_idx, node_table, relation_table)
    out = _mm(s_rows, p_rows, o_rows, W, b.reshape(1, EMB))
    return out.reshape(bsz, esz, EMB)


# speedup vs baseline: 1.1272x; 1.1272x over previous
import functools

import jax
import jax.numpy as jnp
from jax import lax
from jax.experimental import pallas as pl
from jax.experimental.pallas import tpu as pltpu
from jax.experimental.pallas import tpu_sc as plsc

EMB = 64
GRP = 128
NUM_WORKERS = 32
MM_BLK = 2048


NGRP_TILE = 64
CHUNK = 4


def _sc_gather_body(s_idx, p_idx, o_idx, node_tab, rel_tab,
                    s_out, p_out, o_out,
                    idx_s, idx_p, idx_o, buf_s, buf_p, buf_o,
                    gsem_s, gsem_p, gsem_o, ssem_s, ssem_p, ssem_o):
    wid = lax.axis_index("s") * 2 + lax.axis_index("c")
    base = wid * NGRP_TILE
    tabs = ((s_idx, node_tab, s_out, idx_s, buf_s, gsem_s, ssem_s),
            (p_idx, rel_tab, p_out, idx_p, buf_p, gsem_p, ssem_p),
            (o_idx, node_tab, o_out, idx_o, buf_o, gsem_o, ssem_o))

    for (ih, th, oh, iv, bv, gs, ss) in tabs:
        pltpu.sync_copy(ih.at[pl.ds(base, NGRP_TILE)], iv)

    def chunk(c, carry):
        handles = []
        for (ih, th, oh, iv, bv, gs, ss) in tabs:
            @pl.when(c > 0)
            def _drain(oh=oh, bv=bv, ss=ss):
                for b in range(CHUNK):
                    pltpu.make_async_copy(
                        bv.at[b], oh.at[pl.ds(b * GRP, GRP)], ss).wait()
            hs = []
            for b in range(CHUNK):
                hs.append(pltpu.async_copy(
                    th.at[iv.at[c * CHUNK + b]], bv.at[b], gs))
            handles.append(hs)
        for (ih, th, oh, iv, bv, gs, ss), hs in zip(tabs, handles):
            for b in range(CHUNK):
                hs[b].wait()
                g = base + c * CHUNK + b
                pltpu.async_copy(bv.at[b], oh.at[pl.ds(g * GRP, GRP)], ss)
        return carry

    lax.fori_loop(0, NGRP_TILE // CHUNK, chunk, 0)
    for (ih, th, oh, iv, bv, gs, ss) in tabs:
        for b in range(CHUNK):
            pltpu.make_async_copy(
                bv.at[b], oh.at[pl.ds(b * GRP, GRP)], ss).wait()


def _sc_gather(s_idx, p_idx, o_idx, node_tab, rel_tab):
    n = s_idx.shape[0] * GRP
    out_t = jax.ShapeDtypeStruct((n, EMB), jnp.float32)
    mesh = plsc.VectorSubcoreMesh(core_axis_name="c", subcore_axis_name="s")
    idx_t = pltpu.VMEM((NGRP_TILE, GRP), jnp.int32)
    buf_t = pltpu.VMEM((CHUNK, GRP, EMB), jnp.float32)
    f = functools.partial(
        pl.kernel,
        mesh=mesh,
        compiler_params=pltpu.CompilerParams(use_tc_tiling_on_sc=False),
        out_type=(out_t, out_t, out_t),
        scratch_types=[idx_t, idx_t, idx_t, buf_t, buf_t, buf_t]
        + [pltpu.SemaphoreType.DMA] * 6,
    )(_sc_gather_body)
    return f(s_idx, p_idx, o_idx, node_tab, rel_tab)


def _mm_body(s_ref, p_ref, o_ref, w_ref, b_ref, out_ref):
    w = w_ref[...]
    acc = jnp.dot(s_ref[...], w[0:EMB], preferred_element_type=jnp.float32)
    acc = acc + jnp.dot(p_ref[...], w[EMB:2 * EMB],
                        preferred_element_type=jnp.float32)
    acc = acc + jnp.dot(o_ref[...], w[2 * EMB:3 * EMB],
                        preferred_element_type=jnp.float32)
    out_ref[...] = acc + b_ref[...]


def _mm(s_rows, p_rows, o_rows, w, b2d):
    n = s_rows.shape[0]
    grid = (n // MM_BLK,)
    row_spec = pl.BlockSpec((MM_BLK, EMB), lambda i: (i, 0))
    return pl.pallas_call(
        _mm_body,
        grid=grid,
        in_specs=[
            row_spec, row_spec, row_spec,
            pl.BlockSpec((3 * EMB, EMB), lambda i: (0, 0)),
            pl.BlockSpec((1, EMB), lambda i: (0, 0)),
        ],
        out_specs=row_spec,
        out_shape=jax.ShapeDtypeStruct((n, EMB), jnp.float32),
    )(s_rows, p_rows, o_rows, w, b2d)


def kernel(triples, node_table, relation_table, W, b):
    bsz, esz, _ = triples.shape
    n = bsz * esz
    t = triples.reshape(n, 3).astype(jnp.int32)
    s_idx = t[:, 0].reshape(n // GRP, GRP)
    p_idx = t[:, 1].reshape(n // GRP, GRP)
    o_idx = t[:, 2].reshape(n // GRP, GRP)
    s_rows, p_rows, o_rows = _sc_gather(
        s_idx, p_idx, o_idx, node_table, relation_table)
    out = _mm(s_rows, p_rows, o_rows, W, b.reshape(1, EMB))
    return out.reshape(bsz, esz, EMB)


# speedup vs baseline: 1.3158x; 1.1673x over previous
import functools

import jax
import jax.numpy as jnp
from jax import lax
from jax.experimental import pallas as pl
from jax.experimental.pallas import tpu as pltpu
from jax.experimental.pallas import tpu_sc as plsc

EMB = 64
GRP = 128
NUM_WORKERS = 32
NGRP_TILE = 64
CHUNK = 4
MM_BLK = 1024


def _sc_gather_body(s_idx, p_idx, o_idx, node_tab, rel_tab,
                    s_out, p_out, o_out,
                    idx_s, idx_p, idx_o, buf_s, buf_p, buf_o,
                    gsem_s, gsem_p, gsem_o, ssem_s, ssem_p, ssem_o):
    wid = lax.axis_index("s") * 2 + lax.axis_index("c")
    base = wid * NGRP_TILE
    tabs = ((s_idx, node_tab, s_out, idx_s, buf_s, gsem_s, ssem_s),
            (p_idx, rel_tab, p_out, idx_p, buf_p, gsem_p, ssem_p),
            (o_idx, node_tab, o_out, idx_o, buf_o, gsem_o, ssem_o))

    for (ih, th, oh, iv, bv, gs, ss) in tabs:
        pltpu.sync_copy(ih.at[pl.ds(base, NGRP_TILE)], iv)

    def chunk(c, carry):
        handles = []
        for (ih, th, oh, iv, bv, gs, ss) in tabs:
            @pl.when(c > 0)
            def _drain(oh=oh, bv=bv, ss=ss):
                for b in range(CHUNK):
                    pltpu.make_async_copy(bv.at[b], oh.at[b], ss).wait()
            hs = []
            for b in range(CHUNK):
                hs.append(pltpu.async_copy(
                    th.at[iv.at[c * CHUNK + b]], bv.at[b], gs))
            handles.append(hs)
        for (ih, th, oh, iv, bv, gs, ss), hs in zip(tabs, handles):
            for b in range(CHUNK):
                hs[b].wait()
                g = base + c * CHUNK + b
                pltpu.async_copy(bv.at[b], oh.at[g], ss)
        return carry

    lax.fori_loop(0, NGRP_TILE // CHUNK, chunk, 0)
    for (ih, th, oh, iv, bv, gs, ss) in tabs:
        for b in range(CHUNK):
            pltpu.make_async_copy(bv.at[b], oh.at[b], ss).wait()


def _sc_gather(s_idx, p_idx, o_idx, node_tab, rel_tab):
    ngrp = s_idx.shape[0]
    out_t = jax.ShapeDtypeStruct((ngrp, GRP, EMB), jnp.float32)
    mesh = plsc.VectorSubcoreMesh(core_axis_name="c", subcore_axis_name="s")
    idx_t = pltpu.VMEM((NGRP_TILE, GRP), jnp.int32)
    buf_t = pltpu.VMEM((CHUNK, GRP, EMB), jnp.float32)
    f = functools.partial(
        pl.kernel,
        mesh=mesh,
        compiler_params=pltpu.CompilerParams(use_tc_tiling_on_sc=False),
        out_type=(out_t, out_t, out_t),
        scratch_types=[idx_t, idx_t, idx_t, buf_t, buf_t, buf_t]
        + [pltpu.SemaphoreType.DMA] * 6,
    )(_sc_gather_body)
    return f(s_idx, p_idx, o_idx, node_tab, rel_tab)


def _mm_body(s_ref, p_ref, o_ref, ws_ref, wp_ref, wo_ref, b_ref, out_ref):
    acc = jnp.dot(s_ref[...], ws_ref[...], preferred_element_type=jnp.float32)
    acc = acc + jnp.dot(p_ref[...], wp_ref[...],
                        preferred_element_type=jnp.float32)
    acc = acc + jnp.dot(o_ref[...], wo_ref[...],
                        preferred_element_type=jnp.float32)
    out_ref[...] = acc + b_ref[...]


def _mm(s2, p2, o2, wds, wdp, wdo, bb):
    n2 = s2.shape[0]
    grid = (n2 // MM_BLK,)
    row_spec = pl.BlockSpec((MM_BLK, 2 * EMB), lambda i: (i, 0))
    w_spec = pl.BlockSpec((2 * EMB, 2 * EMB), lambda i: (0, 0))
    return pl.pallas_call(
        _mm_body,
        grid=grid,
        in_specs=[row_spec, row_spec, row_spec, w_spec, w_spec, w_spec,
                  pl.BlockSpec((1, 2 * EMB), lambda i: (0, 0))],
        out_specs=row_spec,
        out_shape=jax.ShapeDtypeStruct((n2, 2 * EMB), jnp.float32),
    )(s2, p2, o2, wds, wdp, wdo, bb)


def _blockdiag(wt):
    z = jnp.zeros((EMB, EMB), jnp.float32)
    return jnp.concatenate(
        [jnp.concatenate([wt, z], axis=1),
         jnp.concatenate([z, wt], axis=1)], axis=0)


def kernel(triples, node_table, relation_table, W, b):
    bsz, esz, _ = triples.shape
    n = bsz * esz
    t = triples.reshape(n, 3).astype(jnp.int32)
    s_idx = t[:, 0].reshape(n // GRP, GRP)
    p_idx = t[:, 1].reshape(n // GRP, GRP)
    o_idx = t[:, 2].reshape(n // GRP, GRP)
    s3, p3, o3 = _sc_gather(s_idx, p_idx, o_idx, node_table, relation_table)
    s2 = s3.reshape(n // 2, 2 * EMB)
    p2 = p3.reshape(n // 2, 2 * EMB)
    o2 = o3.reshape(n // 2, 2 * EMB)
    wds = _blockdiag(W[0:EMB])
    wdp = _blockdiag(W[EMB:2 * EMB])
    wdo = _blockdiag(W[2 * EMB:3 * EMB])
    bb = jnp.concatenate([b, b]).reshape(1, 2 * EMB)
    out2 = _mm(s2, p2, o2, wds, wdp, wdo, bb)
    return out2.reshape(bsz, esz, EMB)
